# initial kernel scaffold (unmeasured)
import jax
import jax.numpy as jnp
from jax import lax
from jax.experimental import pallas as pl
from jax.experimental.pallas import tpu as pltpu

N_DEV = 16
N_ROWS = 1024
D_IN = 256
D_OUT = 512
N_EXP = 64
E_LOCAL = 4
ROWS_PER = N_ROWS // N_DEV


def kernel(x, router_W, route_idx, expert_W, shared_W):
    def body(x_ref, rW_ref, idx_ref, eW_ref, sW_ref, out_ref,
             partial_ref, comm_ref, send_sems, recv_sems):
        my = lax.axis_index("i")

        xv = x_ref[...]
        scores = jnp.dot(xv, rW_ref[...],
                         preferred_element_type=jnp.float32)
        m = jnp.max(scores, axis=1, keepdims=True)
        e = jnp.exp(scores - m)
        probs = e / jnp.sum(e, axis=1, keepdims=True)

        sel = idx_ref[...]
        e_iota = lax.broadcasted_iota(jnp.int32, (N_ROWS, N_EXP), 1)
        pv = jnp.sum(jnp.where(e_iota == sel, probs, 0.0),
                     axis=1, keepdims=True)

        xb = xv.astype(jnp.bfloat16)
        partial = jnp.zeros((N_ROWS, D_OUT), jnp.float32)
        for el in range(E_LOCAL):
            eg = E_LOCAL * my + el
            coef = jnp.where(sel == eg, pv, 0.0)
            w = eW_ref[el].astype(jnp.bfloat16)
            y = jnp.dot(xb, w, preferred_element_type=jnp.float32)
            partial = partial + coef * y
        partial_ref[...] = partial.astype(jnp.bfloat16)

        def flow(o):
            t = lax.rem(my + o, N_DEV)
            return pltpu.make_async_remote_copy(
                src_ref=partial_ref.at[pl.ds(t * ROWS_PER, ROWS_PER)],
                dst_ref=comm_ref.at[o],
                send_sem=send_sems.at[o],
                recv_sem=recv_sems.at[o],
                device_id=(t,),
                device_id_type=pl.DeviceIdType.MESH,
            )

        for o in range(1, N_DEV):
            flow(o).start()

        x_my = lax.dynamic_slice(xv, (my * ROWS_PER, 0), (ROWS_PER, D_IN))
        shared_my = jnp.dot(x_my.astype(jnp.bfloat16),
                            sW_ref[...].astype(jnp.bfloat16),
                            preferred_element_type=jnp.float32)
        own = lax.dynamic_slice(partial, (my * ROWS_PER, 0),
                                (ROWS_PER, D_OUT))
        acc = shared_my + own

        for o in range(1, N_DEV):
            w = flow(o)
            w.wait()
            acc = acc + comm_ref[o].astype(jnp.float32)

        out_ref[...] = acc

    return pl.pallas_call(
        body,
        out_shape=jax.ShapeDtypeStruct((ROWS_PER, D_OUT), jnp.float32),
        in_specs=[pl.BlockSpec(memory_space=pltpu.VMEM)] * 5,
        out_specs=pl.BlockSpec(memory_space=pltpu.VMEM),
        scratch_shapes=[
            pltpu.VMEM((N_ROWS, D_OUT), jnp.bfloat16),
            pltpu.VMEM((N_DEV, ROWS_PER, D_OUT), jnp.bfloat16),
            pltpu.SemaphoreType.DMA((N_DEV,)),
            pltpu.SemaphoreType.DMA((N_DEV,)),
        ],
    )(x, router_W, route_idx, expert_W, shared_W)


# baseline (device time: 28662 ns/iter reference)
import jax
import jax.numpy as jnp
from jax import lax
from jax.experimental import pallas as pl
from jax.experimental.pallas import tpu as pltpu

N_DEV = 16
N_ROWS = 1024
D_IN = 256
D_OUT = 512
N_EXP = 64
E_LOCAL = 4
ROWS_PER = N_ROWS // N_DEV


def kernel(x, router_W, route_idx, expert_W, shared_W):
    def body(x_ref, rW_ref, idx_ref, eW_ref, sW_ref, out_ref,
             partial_ref, comm_ref, send_sems, recv_sems):
        my = lax.axis_index("i")

        xv = x_ref[...]
        scores = jnp.dot(xv, rW_ref[...],
                         preferred_element_type=jnp.float32)
        m = jnp.max(scores, axis=1, keepdims=True)
        e = jnp.exp(scores - m)
        probs = e / jnp.sum(e, axis=1, keepdims=True)

        sel = idx_ref[...]
        e_iota = lax.broadcasted_iota(jnp.int32, (N_ROWS, N_EXP), 1)
        pv = jnp.sum(jnp.where(e_iota == sel, probs, 0.0),
                     axis=1, keepdims=True)

        xb = xv.astype(jnp.bfloat16)
        partial = jnp.zeros((N_ROWS, D_OUT), jnp.float32)
        for el in range(E_LOCAL):
            eg = E_LOCAL * my + el
            coef = jnp.where(sel == eg, pv, 0.0)
            w = eW_ref[el].astype(jnp.bfloat16)
            y = jnp.dot(xb, w, preferred_element_type=jnp.float32)
            partial = partial + coef * y
        partial_ref[...] = partial.astype(jnp.bfloat16)

        def flow(o):
            t = lax.rem(my + o, N_DEV)
            return pltpu.make_async_remote_copy(
                src_ref=partial_ref.at[pl.ds(t * ROWS_PER, ROWS_PER)],
                dst_ref=comm_ref.at[o],
                send_sem=send_sems.at[o],
                recv_sem=recv_sems.at[o],
                device_id=(t,),
                device_id_type=pl.DeviceIdType.MESH,
            )

        for o in range(1, N_DEV):
            flow(o).start()

        x_my = x_ref[pl.ds(my * ROWS_PER, ROWS_PER), :]
        shared_my = jnp.dot(x_my.astype(jnp.bfloat16),
                            sW_ref[...].astype(jnp.bfloat16),
                            preferred_element_type=jnp.float32)
        own = partial_ref[pl.ds(my * ROWS_PER, ROWS_PER), :]
        acc = shared_my + own.astype(jnp.float32)

        for o in range(1, N_DEV):
            w = flow(o)
            w.wait()
            acc = acc + comm_ref[o].astype(jnp.float32)

        out_ref[...] = acc

    return pl.pallas_call(
        body,
        out_shape=jax.ShapeDtypeStruct((ROWS_PER, D_OUT), jnp.float32),
        in_specs=[pl.BlockSpec(memory_space=pltpu.VMEM)] * 5,
        out_specs=pl.BlockSpec(memory_space=pltpu.VMEM),
        scratch_shapes=[
            pltpu.VMEM((N_ROWS, D_OUT), jnp.bfloat16),
            pltpu.VMEM((N_DEV, ROWS_PER, D_OUT), jnp.bfloat16),
            pltpu.SemaphoreType.DMA((N_DEV,)),
            pltpu.SemaphoreType.DMA((N_DEV,)),
        ],
    )(x, router_W, route_idx, expert_W, shared_W)
